# baseline (speedup 1.0000x reference)
import functools

import jax
import jax.numpy as jnp
from jax import lax
from jax.experimental import pallas as pl
from jax.experimental.pallas import tpu as pltpu
from jax.experimental.pallas import tpu_sc as plsc

N = 10000
E = 320000
D = 128
DE = 4
F = 128
R = 2 * E

NC, NS, L = 2, 16, 16
NW = NC * NS
BATCH = 128
NCHUNK = 4
EC = E // NCHUNK
RC = 2 * EC
NB_TOT = RC // BATCH
NB_BASE = NB_TOT // NW
NB_EXTRA = NB_TOT - NB_BASE * NW
SC_ITERS = NB_BASE + 1
NSLOT = 4
NSLOT_S = 2
IDX_CAP = SC_ITERS * BATCH
IDX_PAD = (NB_TOT + 1) * BATCH
ROWS_PT = 624
ROWS_TAIL = N - NS * ROWS_PT
EPACK = 128 // DE
N_EDGE_BAND = E // EPACK

_LN_EPS = 1e-6


def _ln(x, g, b):
    mu = jnp.mean(x, axis=-1, keepdims=True)
    xc = x - mu
    var = jnp.mean(xc * xc, axis=-1, keepdims=True)
    inv = lax.rsqrt(var + _LN_EPS)
    return xc * inv * g + b


def _stage1_body(eps_ref, n_ref, sW1_ref, sb1_ref, sg1_ref, sbe1_ref,
                 sW2_ref, sb2_ref, sg2_ref, sbe2_ref, W1a_ref, mb1_ref,
                 h_ref, a_ref):
    n = n_ref[...]
    x = jnp.maximum(jnp.dot(n, sW1_ref[...], preferred_element_type=jnp.float32)
                    + sb1_ref[...], 0.0)
    x = _ln(x, sg1_ref[...], sbe1_ref[...])
    x = jnp.maximum(jnp.dot(x, sW2_ref[...], preferred_element_type=jnp.float32)
                    + sb2_ref[...], 0.0)
    x = _ln(x, sg2_ref[...], sbe2_ref[...])
    h_ref[...] = x * (1.0 + eps_ref[0, 0])
    a_ref[...] = (jnp.dot(n, W1a_ref[...], preferred_element_type=jnp.float32)
                  + mb1_ref[...])


def _stage1(nodes, eps2, sW1, sb1, sg1, sbe1, sW2, sb2, sg2, sbe2, W1a, mb1):
    BN = 2000
    row = pl.BlockSpec((BN, D), lambda i: (i, 0))
    mat = pl.BlockSpec((D, F), lambda i: (0, 0))
    vec = pl.BlockSpec((1, F), lambda i: (0, 0))
    scl = pl.BlockSpec((1, 1), lambda i: (0, 0))
    return pl.pallas_call(
        _stage1_body,
        grid=(N // BN,),
        in_specs=[scl, row, mat, vec, vec, vec, mat, vec, vec, vec, mat, vec],
        out_specs=[row, row],
        out_shape=[jax.ShapeDtypeStruct((N, F), jnp.float32)] * 2,
    )(eps2, nodes, sW1, sb1, sg1, sbe1, sW2, sb2, sg2, sbe2, W1a, mb1)


def _worker_range(wid):
    start = wid * NB_BASE + jnp.minimum(wid, NB_EXTRA)
    nb = jnp.where(wid < NB_EXTRA, NB_BASE + 1, NB_BASE)
    return start, nb


def _sc_gather(table, idx):
    mesh = plsc.VectorSubcoreMesh(core_axis_name="c", subcore_axis_name="s")

    @functools.partial(
        pl.kernel, mesh=mesh,
        out_type=jax.ShapeDtypeStruct((RC, F), jnp.float32),
        scratch_types=[
            pltpu.VMEM((IDX_CAP,), jnp.int32),
            pltpu.VMEM((NSLOT, BATCH, F), jnp.float32),
            pltpu.SemaphoreType.DMA,
            pltpu.SemaphoreType.DMA,
            pltpu.SemaphoreType.DMA,
            pltpu.SemaphoreType.DMA,
            pltpu.SemaphoreType.DMA,
            pltpu.SemaphoreType.DMA,
            pltpu.SemaphoreType.DMA,
            pltpu.SemaphoreType.DMA,
        ],
    )
    def k(table_hbm, idx_hbm, out_hbm, idx_v, rows_v,
          sg0, sg1, sg2, sg3, so0, so1, so2, so3):
        wid = lax.axis_index("s") * NC + lax.axis_index("c")
        start, nb = _worker_range(wid)
        semg = [sg0, sg1, sg2, sg3]
        semo = [so0, so1, so2, so3]
        pltpu.sync_copy(idx_hbm.at[pl.ds(start * BATCH, IDX_CAP)], idx_v)

        def fire(slot, i):
            @pl.when(i < nb)
            def _():
                pltpu.async_copy(
                    table_hbm.at[idx_v.at[pl.ds(i * BATCH, BATCH)]],
                    rows_v.at[slot], semg[slot])

        for j in range(NSLOT):
            fire(j, j)

        def body(g, carry):
            i0 = g * NSLOT
            for j in range(NSLOT):
                i = i0 + j

                @pl.when(i < nb)
                def _(i=i, j=j):
                    pltpu.make_async_copy(
                        table_hbm.at[idx_v.at[pl.ds(i * BATCH, BATCH)]],
                        rows_v.at[j], semg[j]).wait()
                    off = (start + i) * BATCH
                    pltpu.async_copy(rows_v.at[j],
                                     out_hbm.at[pl.ds(off, BATCH)], semo[j])
                    pltpu.make_async_copy(
                        rows_v.at[j], out_hbm.at[pl.ds(off, BATCH)],
                        semo[j]).wait()
                    fire(j, i + NSLOT)

            return carry

        lax.fori_loop(0, -(-SC_ITERS // NSLOT), body, 0)

    return k(table, idx)


def _stage3_body(g_ref, e_ref, wexp_ref, g1_ref, be1_ref,
                 w2_ref, b2_ref, g2_ref, be2_ref, out_ref):
    ep = jnp.dot(e_ref[...], wexp_ref[0], preferred_element_type=jnp.float32)
    x = jnp.maximum(g_ref[...] + ep, 0.0)
    x = _ln(x, g1_ref[...], be1_ref[...])
    x = jnp.maximum(jnp.dot(x, w2_ref[...], preferred_element_type=jnp.float32)
                    + b2_ref[...], 0.0)
    x = _ln(x, g2_ref[...], be2_ref[...])
    out_ref[...] = x


def _stage3(G, ec, Wexp, g1, be1, W2, b2, g2, be2, c):
    BE = 2000
    npb = N_EDGE_BAND // BE
    half = EC // BE
    bands = EC // N_EDGE_BAND
    row = pl.BlockSpec((BE, F), lambda i: (i, 0))
    erow = pl.BlockSpec((BE, F), lambda i: (lax.rem(i, npb), 0))
    wexp = pl.BlockSpec((1, F, F),
                        lambda i: (bands * c + lax.rem(i, half) // npb, 0, 0))
    mat = pl.BlockSpec((F, F), lambda i: (0, 0))
    vec = pl.BlockSpec((1, F), lambda i: (0, 0))
    return pl.pallas_call(
        _stage3_body,
        grid=(RC // BE,),
        in_specs=[row, erow, wexp, vec, vec, mat, vec, vec, vec],
        out_specs=row,
        out_shape=jax.ShapeDtypeStruct((RC, F), jnp.float32),
    )(G, ec, Wexp, g1, be1, W2, b2, g2, be2)


def _sc_scatter(msgs, idx, zeros_nf):
    mesh = plsc.VectorSubcoreMesh(core_axis_name="c", subcore_axis_name="s")

    @functools.partial(
        pl.kernel, mesh=mesh,
        out_type=jax.ShapeDtypeStruct((NC * N, F), jnp.float32),
        scratch_types=[
            pltpu.VMEM((NSLOT_S, BATCH), jnp.int32),
            pltpu.VMEM((NSLOT_S, BATCH, F), jnp.float32),
            pltpu.VMEM_SHARED((N, F), jnp.float32),
            pltpu.SemaphoreType.DMA,
            pltpu.SemaphoreType.DMA,
            pltpu.SemaphoreType.DMA,
            pltpu.SemaphoreType.DMA,
        ],
    )
    def k(msgs_hbm, idx_hbm, zeros_hbm, out_hbm, idx_vs, rows_v,
          acc_sh, si0, si1, sm0, sm1):
        cid = lax.axis_index("c")
        sid = lax.axis_index("s")
        wid = sid * NC + cid
        start, nb = _worker_range(wid)
        semi = [si0, si1]
        semm = [sm0, sm1]
        pltpu.sync_copy(zeros_hbm.at[pl.ds(sid * ROWS_PT, ROWS_PT)],
                        acc_sh.at[pl.ds(sid * ROWS_PT, ROWS_PT)])

        @pl.when(sid == NS - 1)
        def _zero_tail():
            pltpu.sync_copy(zeros_hbm.at[pl.ds(NS * ROWS_PT, ROWS_TAIL)],
                            acc_sh.at[pl.ds(NS * ROWS_PT, ROWS_TAIL)])

        plsc.subcore_barrier()

        def fire(slot, i):
            @pl.when(i < nb)
            def _():
                off = (start + i) * BATCH
                pltpu.async_copy(idx_hbm.at[pl.ds(off, BATCH)],
                                 idx_vs.at[slot], semi[slot])
                pltpu.async_copy(msgs_hbm.at[pl.ds(off, BATCH)],
                                 rows_v.at[slot], semm[slot])

        for j in range(NSLOT_S):
            fire(j, j)

        def body(g, carry):
            i0 = g * NSLOT_S
            for j in range(NSLOT_S):
                i = i0 + j

                @pl.when(i < nb)
                def _(i=i, j=j):
                    off = (start + i) * BATCH
                    pltpu.make_async_copy(idx_hbm.at[pl.ds(off, BATCH)],
                                          idx_vs.at[j], semi[j]).wait()
                    pltpu.make_async_copy(msgs_hbm.at[pl.ds(off, BATCH)],
                                          rows_v.at[j], semm[j]).wait()
                    pltpu.sync_copy(rows_v.at[j], acc_sh.at[idx_vs.at[j]],
                                    add=True)
                    fire(j, i + NSLOT_S)

            return carry

        lax.fori_loop(0, -(-SC_ITERS // NSLOT_S), body, 0)
        plsc.subcore_barrier()
        pltpu.sync_copy(acc_sh.at[pl.ds(sid * ROWS_PT, ROWS_PT)],
                        out_hbm.at[pl.ds(cid * N + sid * ROWS_PT, ROWS_PT)])

        @pl.when(sid == NS - 1)
        def _out_tail():
            pltpu.sync_copy(acc_sh.at[pl.ds(NS * ROWS_PT, ROWS_TAIL)],
                            out_hbm.at[pl.ds(cid * N + NS * ROWS_PT, ROWS_TAIL)])

    return k(msgs, idx, zeros_nf)


def _stage5_body(*refs):
    h_ref = refs[0]
    p_refs = refs[1:1 + 2 * NCHUNK]
    (W1_ref, b1_ref, g1_ref, be1_ref, W2_ref, b2_ref, g2_ref, be2_ref,
     out_ref) = refs[1 + 2 * NCHUNK:]
    x = h_ref[...]
    for p in p_refs:
        x = x + p[...]
    x = jnp.maximum(jnp.dot(x, W1_ref[...], preferred_element_type=jnp.float32)
                    + b1_ref[...], 0.0)
    x = _ln(x, g1_ref[...], be1_ref[...])
    x = jnp.maximum(jnp.dot(x, W2_ref[...], preferred_element_type=jnp.float32)
                    + b2_ref[...], 0.0)
    x = _ln(x, g2_ref[...], be2_ref[...])
    out_ref[...] = x


def _stage5(h, parts, W1, b1, g1, be1, W2, b2, g2, be2):
    BN = 2000
    row = pl.BlockSpec((BN, F), lambda i: (i, 0))
    p0 = pl.BlockSpec((BN, F), lambda i: (i, 0))
    p1 = pl.BlockSpec((BN, F), lambda i: (i + N // BN, 0))
    mat = pl.BlockSpec((F, F), lambda i: (0, 0))
    vec = pl.BlockSpec((1, F), lambda i: (0, 0))
    part_args = [p for part in parts for p in (part, part)]
    part_specs = [s for _ in parts for s in (p0, p1)]
    return pl.pallas_call(
        _stage5_body,
        grid=(N // BN,),
        in_specs=[row] + part_specs
                 + [mat, vec, vec, vec, mat, vec, vec, vec],
        out_specs=row,
        out_shape=jax.ShapeDtypeStruct((N, F), jnp.float32),
    )(h, *part_args, W1, b1, g1, be1, W2, b2, g2, be2)


def kernel(nodes, edges, eps_const,
           msg_W1, msg_b1, msg_g1, msg_be1, msg_W2, msg_b2, msg_g2, msg_be2,
           self_W1, self_b1, self_g1, self_be1, self_W2, self_b2, self_g2,
           self_be2, node_W1, node_b1, node_g1, node_be1, node_W2, node_b2,
           node_g2, node_be2, senders, receivers):
    W1a = msg_W1[:D]
    W1b = msg_W1[D:]
    pad = jnp.zeros((IDX_PAD - RC,), jnp.int32)
    eps2 = eps_const.reshape(1, 1)
    zeros_nf = jnp.zeros((N, F), jnp.float32)
    v = lambda a: a.reshape(1, F)

    sp = senders.reshape(N_EDGE_BAND, EPACK).T.reshape(-1)
    rp = receivers.reshape(N_EDGE_BAND, EPACK).T.reshape(-1)
    ec = edges.reshape(N_EDGE_BAND, EPACK * DE)
    Wexp = jnp.stack([jnp.pad(W1b, ((DE * j, F - DE * (j + 1)), (0, 0)))
                      for j in range(EPACK)])

    h_scaled, A = _stage1(nodes, eps2, self_W1, v(self_b1), v(self_g1),
                          v(self_be1), self_W2, v(self_b2), v(self_g2),
                          v(self_be2), W1a, v(msg_b1))
    parts = []
    for c in range(NCHUNK):
        a, b = c * EC, (c + 1) * EC
        s_idx = jnp.concatenate([sp[a:b], rp[a:b], pad])
        r_idx = jnp.concatenate([rp[a:b], sp[a:b], pad])
        G = _sc_gather(A, s_idx)
        msgs = _stage3(G, ec, Wexp, v(msg_g1), v(msg_be1), msg_W2,
                       v(msg_b2), v(msg_g2), v(msg_be2), c)
        parts.append(_sc_scatter(msgs, r_idx, zeros_nf))
    out = _stage5(h_scaled, parts, node_W1, v(node_b1), v(node_g1),
                  v(node_be1), node_W2, v(node_b2), v(node_g2), v(node_be2))
    return out


# speedup vs baseline: 1.2783x; 1.2783x over previous
import functools

import jax
import jax.numpy as jnp
from jax import lax
from jax.experimental import pallas as pl
from jax.experimental.pallas import tpu as pltpu
from jax.experimental.pallas import tpu_sc as plsc

N = 10000
E = 320000
D = 128
DE = 4
F = 128
R = 2 * E

NC, NS, L = 2, 16, 16
NW = NC * NS
BATCH = 128
NCHUNK = 4
EC = E // NCHUNK
RC = 2 * EC
NB_TOT = RC // BATCH
NB_BASE = NB_TOT // NW
NB_EXTRA = NB_TOT - NB_BASE * NW
SC_ITERS = NB_BASE + 1
NSLOT = 4
NSLOT_S = 2
IDX_CAP = SC_ITERS * BATCH
IDX_PAD = (NB_TOT + 1) * BATCH
ROWS_PT = 624
ROWS_TAIL = N - NS * ROWS_PT
EPACK = 128 // DE
N_EDGE_BAND = E // EPACK

_LN_EPS = 1e-6


def _ln(x, g, b):
    mu = jnp.mean(x, axis=-1, keepdims=True)
    xc = x - mu
    var = jnp.mean(xc * xc, axis=-1, keepdims=True)
    inv = lax.rsqrt(var + _LN_EPS)
    return xc * inv * g + b


def _stage1_body(eps_ref, n_ref, sW1_ref, sb1_ref, sg1_ref, sbe1_ref,
                 sW2_ref, sb2_ref, sg2_ref, sbe2_ref, W1a_ref, mb1_ref,
                 h_ref, a_ref):
    n = n_ref[...]
    x = jnp.maximum(jnp.dot(n, sW1_ref[...], preferred_element_type=jnp.float32)
                    + sb1_ref[...], 0.0)
    x = _ln(x, sg1_ref[...], sbe1_ref[...])
    x = jnp.maximum(jnp.dot(x, sW2_ref[...], preferred_element_type=jnp.float32)
                    + sb2_ref[...], 0.0)
    x = _ln(x, sg2_ref[...], sbe2_ref[...])
    h_ref[...] = x * (1.0 + eps_ref[0, 0])
    a_ref[...] = (jnp.dot(n, W1a_ref[...], preferred_element_type=jnp.float32)
                  + mb1_ref[...])


def _stage1(nodes, eps2, sW1, sb1, sg1, sbe1, sW2, sb2, sg2, sbe2, W1a, mb1):
    BN = 2000
    row = pl.BlockSpec((BN, D), lambda i: (i, 0))
    mat = pl.BlockSpec((D, F), lambda i: (0, 0))
    vec = pl.BlockSpec((1, F), lambda i: (0, 0))
    scl = pl.BlockSpec((1, 1), lambda i: (0, 0))
    return pl.pallas_call(
        _stage1_body,
        grid=(N // BN,),
        in_specs=[scl, row, mat, vec, vec, vec, mat, vec, vec, vec, mat, vec],
        out_specs=[row, row],
        out_shape=[jax.ShapeDtypeStruct((N, F), jnp.float32)] * 2,
    )(eps2, nodes, sW1, sb1, sg1, sbe1, sW2, sb2, sg2, sbe2, W1a, mb1)


def _worker_range(wid):
    start = wid * NB_BASE + jnp.minimum(wid, NB_EXTRA)
    nb = jnp.where(wid < NB_EXTRA, NB_BASE + 1, NB_BASE)
    return start, nb


def _sc_gather(table, idx):
    mesh = plsc.VectorSubcoreMesh(core_axis_name="c", subcore_axis_name="s")

    @functools.partial(
        pl.kernel, mesh=mesh,
        out_type=jax.ShapeDtypeStruct((RC, F), jnp.float32),
        scratch_types=[
            pltpu.VMEM((IDX_CAP,), jnp.int32),
            pltpu.VMEM((NSLOT, BATCH, F), jnp.float32),
            pltpu.SemaphoreType.DMA,
            pltpu.SemaphoreType.DMA,
            pltpu.SemaphoreType.DMA,
            pltpu.SemaphoreType.DMA,
            pltpu.SemaphoreType.DMA,
            pltpu.SemaphoreType.DMA,
            pltpu.SemaphoreType.DMA,
            pltpu.SemaphoreType.DMA,
        ],
    )
    def k(table_hbm, idx_hbm, out_hbm, idx_v, rows_v,
          sg0, sg1, sg2, sg3, so0, so1, so2, so3):
        wid = lax.axis_index("s") * NC + lax.axis_index("c")
        start, nb = _worker_range(wid)
        semg = [sg0, sg1, sg2, sg3]
        semo = [so0, so1, so2, so3]
        pltpu.sync_copy(idx_hbm.at[pl.ds(start * BATCH, IDX_CAP)], idx_v)

        def fire(slot, i):
            @pl.when(i < nb)
            def _():
                pltpu.async_copy(
                    table_hbm.at[idx_v.at[pl.ds(i * BATCH, BATCH)]],
                    rows_v.at[slot], semg[slot])

        for j in range(NSLOT):
            fire(j, j)

        def body(g, carry):
            i0 = g * NSLOT
            for j in range(NSLOT):
                i = i0 + j

                @pl.when(i < nb)
                def _(i=i, j=j):
                    pltpu.make_async_copy(
                        table_hbm.at[idx_v.at[pl.ds(i * BATCH, BATCH)]],
                        rows_v.at[j], semg[j]).wait()
                    off = (start + i) * BATCH
                    pltpu.async_copy(rows_v.at[j],
                                     out_hbm.at[pl.ds(off, BATCH)], semo[j])
                    pltpu.make_async_copy(
                        rows_v.at[j], out_hbm.at[pl.ds(off, BATCH)],
                        semo[j]).wait()
                    fire(j, i + NSLOT)

            return carry

        lax.fori_loop(0, -(-SC_ITERS // NSLOT), body, 0)

    return k(table, idx)


def _stage3_body(g_ref, e_ref, w1b_ref, g1_ref, be1_ref,
                 w2_ref, b2_ref, g2_ref, be2_ref, out_ref):
    ep = jnp.dot(e_ref[...], w1b_ref[...], preferred_element_type=jnp.float32)
    x = jnp.maximum(g_ref[...] + ep, 0.0)
    x = _ln(x, g1_ref[...], be1_ref[...])
    x = jnp.maximum(jnp.dot(x, w2_ref[...], preferred_element_type=jnp.float32)
                    + b2_ref[...], 0.0)
    x = _ln(x, g2_ref[...], be2_ref[...])
    out_ref[...] = x


def _stage3(G, edges, W1b, g1, be1, W2, b2, g2, be2, c):
    BE = 3200
    nE = EC // BE
    row = pl.BlockSpec((BE, F), lambda i: (lax.rem(i, 2) * nE + i // 2, 0))
    erow = pl.BlockSpec((BE, DE), lambda i: (c * nE + i // 2, 0))
    mat = pl.BlockSpec((F, F), lambda i: (0, 0))
    w1b = pl.BlockSpec((DE, F), lambda i: (0, 0))
    vec = pl.BlockSpec((1, F), lambda i: (0, 0))
    return pl.pallas_call(
        _stage3_body,
        grid=(RC // BE,),
        in_specs=[row, erow, w1b, vec, vec, mat, vec, vec, vec],
        out_specs=row,
        out_shape=jax.ShapeDtypeStruct((RC, F), jnp.float32),
    )(G, edges, W1b, g1, be1, W2, b2, g2, be2)


def _sc_scatter(msgs, idx, zeros_nf):
    mesh = plsc.VectorSubcoreMesh(core_axis_name="c", subcore_axis_name="s")

    @functools.partial(
        pl.kernel, mesh=mesh,
        out_type=jax.ShapeDtypeStruct((NC * N, F), jnp.float32),
        scratch_types=[
            pltpu.VMEM((NSLOT_S, BATCH), jnp.int32),
            pltpu.VMEM((NSLOT_S, BATCH, F), jnp.float32),
            pltpu.VMEM_SHARED((N, F), jnp.float32),
            pltpu.SemaphoreType.DMA,
            pltpu.SemaphoreType.DMA,
            pltpu.SemaphoreType.DMA,
            pltpu.SemaphoreType.DMA,
        ],
    )
    def k(msgs_hbm, idx_hbm, zeros_hbm, out_hbm, idx_vs, rows_v,
          acc_sh, si0, si1, sm0, sm1):
        cid = lax.axis_index("c")
        sid = lax.axis_index("s")
        wid = sid * NC + cid
        start, nb = _worker_range(wid)
        semi = [si0, si1]
        semm = [sm0, sm1]
        pltpu.sync_copy(zeros_hbm.at[pl.ds(sid * ROWS_PT, ROWS_PT)],
                        acc_sh.at[pl.ds(sid * ROWS_PT, ROWS_PT)])

        @pl.when(sid == NS - 1)
        def _zero_tail():
            pltpu.sync_copy(zeros_hbm.at[pl.ds(NS * ROWS_PT, ROWS_TAIL)],
                            acc_sh.at[pl.ds(NS * ROWS_PT, ROWS_TAIL)])

        plsc.subcore_barrier()

        def fire(slot, i):
            @pl.when(i < nb)
            def _():
                off = (start + i) * BATCH
                pltpu.async_copy(idx_hbm.at[pl.ds(off, BATCH)],
                                 idx_vs.at[slot], semi[slot])
                pltpu.async_copy(msgs_hbm.at[pl.ds(off, BATCH)],
                                 rows_v.at[slot], semm[slot])

        for j in range(NSLOT_S):
            fire(j, j)

        def body(g, carry):
            i0 = g * NSLOT_S
            for j in range(NSLOT_S):
                i = i0 + j

                @pl.when(i < nb)
                def _(i=i, j=j):
                    off = (start + i) * BATCH
                    pltpu.make_async_copy(idx_hbm.at[pl.ds(off, BATCH)],
                                          idx_vs.at[j], semi[j]).wait()
                    pltpu.make_async_copy(msgs_hbm.at[pl.ds(off, BATCH)],
                                          rows_v.at[j], semm[j]).wait()
                    pltpu.sync_copy(rows_v.at[j], acc_sh.at[idx_vs.at[j]],
                                    add=True)
                    fire(j, i + NSLOT_S)

            return carry

        lax.fori_loop(0, -(-SC_ITERS // NSLOT_S), body, 0)
        plsc.subcore_barrier()
        pltpu.sync_copy(acc_sh.at[pl.ds(sid * ROWS_PT, ROWS_PT)],
                        out_hbm.at[pl.ds(cid * N + sid * ROWS_PT, ROWS_PT)])

        @pl.when(sid == NS - 1)
        def _out_tail():
            pltpu.sync_copy(acc_sh.at[pl.ds(NS * ROWS_PT, ROWS_TAIL)],
                            out_hbm.at[pl.ds(cid * N + NS * ROWS_PT, ROWS_TAIL)])

    return k(msgs, idx, zeros_nf)


def _stage5_body(*refs):
    h_ref = refs[0]
    p_refs = refs[1:1 + 2 * NCHUNK]
    (W1_ref, b1_ref, g1_ref, be1_ref, W2_ref, b2_ref, g2_ref, be2_ref,
     out_ref) = refs[1 + 2 * NCHUNK:]
    x = h_ref[...]
    for p in p_refs:
        x = x + p[...]
    x = jnp.maximum(jnp.dot(x, W1_ref[...], preferred_element_type=jnp.float32)
                    + b1_ref[...], 0.0)
    x = _ln(x, g1_ref[...], be1_ref[...])
    x = jnp.maximum(jnp.dot(x, W2_ref[...], preferred_element_type=jnp.float32)
                    + b2_ref[...], 0.0)
    x = _ln(x, g2_ref[...], be2_ref[...])
    out_ref[...] = x


def _stage5(h, parts, W1, b1, g1, be1, W2, b2, g2, be2):
    BN = 2000
    row = pl.BlockSpec((BN, F), lambda i: (i, 0))
    p0 = pl.BlockSpec((BN, F), lambda i: (i, 0))
    p1 = pl.BlockSpec((BN, F), lambda i: (i + N // BN, 0))
    mat = pl.BlockSpec((F, F), lambda i: (0, 0))
    vec = pl.BlockSpec((1, F), lambda i: (0, 0))
    part_args = [p for part in parts for p in (part, part)]
    part_specs = [s for _ in parts for s in (p0, p1)]
    return pl.pallas_call(
        _stage5_body,
        grid=(N // BN,),
        in_specs=[row] + part_specs
                 + [mat, vec, vec, vec, mat, vec, vec, vec],
        out_specs=row,
        out_shape=jax.ShapeDtypeStruct((N, F), jnp.float32),
    )(h, *part_args, W1, b1, g1, be1, W2, b2, g2, be2)


def kernel(nodes, edges, eps_const,
           msg_W1, msg_b1, msg_g1, msg_be1, msg_W2, msg_b2, msg_g2, msg_be2,
           self_W1, self_b1, self_g1, self_be1, self_W2, self_b2, self_g2,
           self_be2, node_W1, node_b1, node_g1, node_be1, node_W2, node_b2,
           node_g2, node_be2, senders, receivers):
    W1a = msg_W1[:D]
    W1b = msg_W1[D:]
    pad = jnp.zeros((IDX_PAD - RC,), jnp.int32)
    eps2 = eps_const.reshape(1, 1)
    zeros_nf = jnp.zeros((N, F), jnp.float32)
    v = lambda a: a.reshape(1, F)

    h_scaled, A = _stage1(nodes, eps2, self_W1, v(self_b1), v(self_g1),
                          v(self_be1), self_W2, v(self_b2), v(self_g2),
                          v(self_be2), W1a, v(msg_b1))
    parts = []
    for c in range(NCHUNK):
        a, b = c * EC, (c + 1) * EC
        s_idx = jnp.concatenate([senders[a:b], receivers[a:b], pad])
        r_idx = jnp.concatenate([receivers[a:b], senders[a:b], pad])
        G = _sc_gather(A, s_idx)
        msgs = _stage3(G, edges, W1b, v(msg_g1), v(msg_be1), msg_W2,
                       v(msg_b2), v(msg_g2), v(msg_be2), c)
        parts.append(_sc_scatter(msgs, r_idx, zeros_nf))
    out = _stage5(h_scaled, parts, node_W1, v(node_b1), v(node_g1),
                  v(node_be1), node_W2, v(node_b2), v(node_g2), v(node_be2))
    return out


# speedup vs baseline: 1.3100x; 1.0248x over previous
import functools

import jax
import jax.numpy as jnp
from jax import lax
from jax.experimental import pallas as pl
from jax.experimental.pallas import tpu as pltpu
from jax.experimental.pallas import tpu_sc as plsc

N = 10000
E = 320000
D = 128
DE = 4
F = 128
R = 2 * E

NC, NS, L = 2, 16, 16
NW = NC * NS
BATCH = 128
NCHUNK = 4
EC = E // NCHUNK
RC = 2 * EC
NB_TOT = RC // BATCH
NB_BASE = NB_TOT // NW
NB_EXTRA = NB_TOT - NB_BASE * NW
SC_ITERS = NB_BASE + 1
NSLOT = 4
NSLOT_S = 2
IDX_CAP = SC_ITERS * BATCH
IDX_PAD = (NB_TOT + 1) * BATCH
ROWS_PT = 624
ROWS_TAIL = N - NS * ROWS_PT
EPACK = 128 // DE
N_EDGE_BAND = E // EPACK

_LN_EPS = 1e-6


def _ln(x, g, b):
    mu = jnp.mean(x, axis=-1, keepdims=True)
    xc = x - mu
    var = jnp.mean(xc * xc, axis=-1, keepdims=True)
    inv = lax.rsqrt(var + _LN_EPS)
    return xc * inv * g + b


def _stage1_body(eps_ref, n_ref, sW1_ref, sb1_ref, sg1_ref, sbe1_ref,
                 sW2_ref, sb2_ref, sg2_ref, sbe2_ref, W1a_ref, mb1_ref,
                 h_ref, a_ref):
    n = n_ref[...]
    x = jnp.maximum(jnp.dot(n, sW1_ref[...], preferred_element_type=jnp.float32)
                    + sb1_ref[...], 0.0)
    x = _ln(x, sg1_ref[...], sbe1_ref[...])
    x = jnp.maximum(jnp.dot(x, sW2_ref[...], preferred_element_type=jnp.float32)
                    + sb2_ref[...], 0.0)
    x = _ln(x, sg2_ref[...], sbe2_ref[...])
    h_ref[...] = x * (1.0 + eps_ref[0, 0])
    a_ref[...] = (jnp.dot(n, W1a_ref[...], preferred_element_type=jnp.float32)
                  + mb1_ref[...])


def _stage1(nodes, eps2, sW1, sb1, sg1, sbe1, sW2, sb2, sg2, sbe2, W1a, mb1):
    BN = 2000
    row = pl.BlockSpec((BN, D), lambda i: (i, 0))
    mat = pl.BlockSpec((D, F), lambda i: (0, 0))
    vec = pl.BlockSpec((1, F), lambda i: (0, 0))
    scl = pl.BlockSpec((1, 1), lambda i: (0, 0))
    return pl.pallas_call(
        _stage1_body,
        grid=(N // BN,),
        in_specs=[scl, row, mat, vec, vec, vec, mat, vec, vec, vec, mat, vec],
        out_specs=[row, row],
        out_shape=[jax.ShapeDtypeStruct((N, F), jnp.float32)] * 2,
    )(eps2, nodes, sW1, sb1, sg1, sbe1, sW2, sb2, sg2, sbe2, W1a, mb1)


def _worker_range(wid):
    start = wid * NB_BASE + jnp.minimum(wid, NB_EXTRA)
    nb = jnp.where(wid < NB_EXTRA, NB_BASE + 1, NB_BASE)
    return start, nb


def _sc_gather(table, idx):
    mesh = plsc.VectorSubcoreMesh(core_axis_name="c", subcore_axis_name="s")

    @functools.partial(
        pl.kernel, mesh=mesh,
        out_type=jax.ShapeDtypeStruct((RC, F), jnp.float32),
        scratch_types=[
            pltpu.VMEM((IDX_CAP,), jnp.int32),
            pltpu.VMEM((NSLOT, BATCH, F), jnp.float32),
            pltpu.SemaphoreType.DMA,
            pltpu.SemaphoreType.DMA,
            pltpu.SemaphoreType.DMA,
            pltpu.SemaphoreType.DMA,
            pltpu.SemaphoreType.DMA,
            pltpu.SemaphoreType.DMA,
            pltpu.SemaphoreType.DMA,
            pltpu.SemaphoreType.DMA,
        ],
    )
    def k(table_hbm, idx_hbm, out_hbm, idx_v, rows_v,
          sg0, sg1, sg2, sg3, so0, so1, so2, so3):
        wid = lax.axis_index("s") * NC + lax.axis_index("c")
        start, nb = _worker_range(wid)
        semg = [sg0, sg1, sg2, sg3]
        semo = [so0, so1, so2, so3]
        pltpu.sync_copy(idx_hbm.at[pl.ds(start * BATCH, IDX_CAP)], idx_v)

        def fire(slot, i):
            @pl.when(i < nb)
            def _():
                pltpu.async_copy(
                    table_hbm.at[idx_v.at[pl.ds(i * BATCH, BATCH)]],
                    rows_v.at[slot], semg[slot])

        for j in range(NSLOT):
            fire(j, j)

        def body(g, carry):
            i0 = g * NSLOT
            for j in range(NSLOT):
                i = i0 + j

                @pl.when(i < nb)
                def _(i=i, j=j):
                    pltpu.make_async_copy(
                        table_hbm.at[idx_v.at[pl.ds(i * BATCH, BATCH)]],
                        rows_v.at[j], semg[j]).wait()
                    off = (start + i) * BATCH
                    pltpu.async_copy(rows_v.at[j],
                                     out_hbm.at[pl.ds(off, BATCH)], semo[j])
                    pltpu.make_async_copy(
                        rows_v.at[j], out_hbm.at[pl.ds(off, BATCH)],
                        semo[j]).wait()
                    fire(j, i + NSLOT)

            return carry

        lax.fori_loop(0, -(-SC_ITERS // NSLOT), body, 0)

    return k(table, idx)


def _stage3_body(g_ref, e_ref, w1b_ref, g1_ref, be1_ref,
                 w2_ref, b2_ref, g2_ref, be2_ref, out_ref):
    ep = jnp.dot(e_ref[...], w1b_ref[...], preferred_element_type=jnp.float32)
    x = jnp.maximum(g_ref[...] + ep, 0.0)
    x = _ln(x, g1_ref[...], be1_ref[...])
    x = jnp.maximum(jnp.dot(x, w2_ref[...], preferred_element_type=jnp.float32)
                    + b2_ref[...], 0.0)
    x = _ln(x, g2_ref[...], be2_ref[...])
    out_ref[...] = x


def _stage3(G, edges, W1b, g1, be1, W2, b2, g2, be2, c):
    BE = 5000
    nE = EC // BE
    row = pl.BlockSpec((BE, F), lambda i: (lax.rem(i, 2) * nE + i // 2, 0))
    erow = pl.BlockSpec((BE, DE), lambda i: (c * nE + i // 2, 0))
    mat = pl.BlockSpec((F, F), lambda i: (0, 0))
    w1b = pl.BlockSpec((DE, F), lambda i: (0, 0))
    vec = pl.BlockSpec((1, F), lambda i: (0, 0))
    return pl.pallas_call(
        _stage3_body,
        grid=(RC // BE,),
        in_specs=[row, erow, w1b, vec, vec, mat, vec, vec, vec],
        out_specs=row,
        out_shape=jax.ShapeDtypeStruct((RC, F), jnp.float32),
    )(G, edges, W1b, g1, be1, W2, b2, g2, be2)


def _sc_scatter(msgs, idx, zeros_nf):
    mesh = plsc.VectorSubcoreMesh(core_axis_name="c", subcore_axis_name="s")

    @functools.partial(
        pl.kernel, mesh=mesh,
        out_type=jax.ShapeDtypeStruct((NC * N, F), jnp.float32),
        scratch_types=[
            pltpu.VMEM((NSLOT_S, BATCH), jnp.int32),
            pltpu.VMEM((NSLOT_S, BATCH, F), jnp.float32),
            pltpu.VMEM_SHARED((N, F), jnp.float32),
            pltpu.SemaphoreType.DMA,
            pltpu.SemaphoreType.DMA,
            pltpu.SemaphoreType.DMA,
            pltpu.SemaphoreType.DMA,
        ],
    )
    def k(msgs_hbm, idx_hbm, zeros_hbm, out_hbm, idx_vs, rows_v,
          acc_sh, si0, si1, sm0, sm1):
        cid = lax.axis_index("c")
        sid = lax.axis_index("s")
        wid = sid * NC + cid
        start, nb = _worker_range(wid)
        semi = [si0, si1]
        semm = [sm0, sm1]
        pltpu.sync_copy(zeros_hbm.at[pl.ds(sid * ROWS_PT, ROWS_PT)],
                        acc_sh.at[pl.ds(sid * ROWS_PT, ROWS_PT)])

        @pl.when(sid == NS - 1)
        def _zero_tail():
            pltpu.sync_copy(zeros_hbm.at[pl.ds(NS * ROWS_PT, ROWS_TAIL)],
                            acc_sh.at[pl.ds(NS * ROWS_PT, ROWS_TAIL)])

        plsc.subcore_barrier()

        def fire(slot, i):
            @pl.when(i < nb)
            def _():
                off = (start + i) * BATCH
                pltpu.async_copy(idx_hbm.at[pl.ds(off, BATCH)],
                                 idx_vs.at[slot], semi[slot])
                pltpu.async_copy(msgs_hbm.at[pl.ds(off, BATCH)],
                                 rows_v.at[slot], semm[slot])

        for j in range(NSLOT_S):
            fire(j, j)

        def body(g, carry):
            i0 = g * NSLOT_S
            for j in range(NSLOT_S):
                i = i0 + j

                @pl.when(i < nb)
                def _(i=i, j=j):
                    off = (start + i) * BATCH
                    pltpu.make_async_copy(idx_hbm.at[pl.ds(off, BATCH)],
                                          idx_vs.at[j], semi[j]).wait()
                    pltpu.make_async_copy(msgs_hbm.at[pl.ds(off, BATCH)],
                                          rows_v.at[j], semm[j]).wait()
                    pltpu.sync_copy(rows_v.at[j], acc_sh.at[idx_vs.at[j]],
                                    add=True)
                    fire(j, i + NSLOT_S)

            return carry

        lax.fori_loop(0, -(-SC_ITERS // NSLOT_S), body, 0)
        plsc.subcore_barrier()
        pltpu.sync_copy(acc_sh.at[pl.ds(sid * ROWS_PT, ROWS_PT)],
                        out_hbm.at[pl.ds(cid * N + sid * ROWS_PT, ROWS_PT)])

        @pl.when(sid == NS - 1)
        def _out_tail():
            pltpu.sync_copy(acc_sh.at[pl.ds(NS * ROWS_PT, ROWS_TAIL)],
                            out_hbm.at[pl.ds(cid * N + NS * ROWS_PT, ROWS_TAIL)])

    return k(msgs, idx, zeros_nf)


def _stage5_body(*refs):
    h_ref = refs[0]
    p_refs = refs[1:1 + 2 * NCHUNK]
    (W1_ref, b1_ref, g1_ref, be1_ref, W2_ref, b2_ref, g2_ref, be2_ref,
     out_ref) = refs[1 + 2 * NCHUNK:]
    x = h_ref[...]
    for p in p_refs:
        x = x + p[...]
    x = jnp.maximum(jnp.dot(x, W1_ref[...], preferred_element_type=jnp.float32)
                    + b1_ref[...], 0.0)
    x = _ln(x, g1_ref[...], be1_ref[...])
    x = jnp.maximum(jnp.dot(x, W2_ref[...], preferred_element_type=jnp.float32)
                    + b2_ref[...], 0.0)
    x = _ln(x, g2_ref[...], be2_ref[...])
    out_ref[...] = x


def _stage5(h, parts, W1, b1, g1, be1, W2, b2, g2, be2):
    BN = 2000
    row = pl.BlockSpec((BN, F), lambda i: (i, 0))
    p0 = pl.BlockSpec((BN, F), lambda i: (i, 0))
    p1 = pl.BlockSpec((BN, F), lambda i: (i + N // BN, 0))
    mat = pl.BlockSpec((F, F), lambda i: (0, 0))
    vec = pl.BlockSpec((1, F), lambda i: (0, 0))
    part_args = [p for part in parts for p in (part, part)]
    part_specs = [s for _ in parts for s in (p0, p1)]
    return pl.pallas_call(
        _stage5_body,
        grid=(N // BN,),
        in_specs=[row] + part_specs
                 + [mat, vec, vec, vec, mat, vec, vec, vec],
        out_specs=row,
        out_shape=jax.ShapeDtypeStruct((N, F), jnp.float32),
    )(h, *part_args, W1, b1, g1, be1, W2, b2, g2, be2)


def kernel(nodes, edges, eps_const,
           msg_W1, msg_b1, msg_g1, msg_be1, msg_W2, msg_b2, msg_g2, msg_be2,
           self_W1, self_b1, self_g1, self_be1, self_W2, self_b2, self_g2,
           self_be2, node_W1, node_b1, node_g1, node_be1, node_W2, node_b2,
           node_g2, node_be2, senders, receivers):
    W1a = msg_W1[:D]
    W1b = msg_W1[D:]
    pad = jnp.zeros((IDX_PAD - RC,), jnp.int32)
    eps2 = eps_const.reshape(1, 1)
    zeros_nf = jnp.zeros((N, F), jnp.float32)
    v = lambda a: a.reshape(1, F)

    h_scaled, A = _stage1(nodes, eps2, self_W1, v(self_b1), v(self_g1),
                          v(self_be1), self_W2, v(self_b2), v(self_g2),
                          v(self_be2), W1a, v(msg_b1))
    parts = []
    for c in range(NCHUNK):
        a, b = c * EC, (c + 1) * EC
        s_idx = jnp.concatenate([senders[a:b], receivers[a:b], pad])
        r_idx = jnp.concatenate([receivers[a:b], senders[a:b], pad])
        G = _sc_gather(A, s_idx)
        msgs = _stage3(G, edges, W1b, v(msg_g1), v(msg_be1), msg_W2,
                       v(msg_b2), v(msg_g2), v(msg_be2), c)
        parts.append(_sc_scatter(msgs, r_idx, zeros_nf))
    out = _stage5(h_scaled, parts, node_W1, v(node_b1), v(node_g1),
                  v(node_be1), node_W2, v(node_b2), v(node_g2), v(node_be2))
    return out


# speedup vs baseline: 1.3118x; 1.0013x over previous
import functools

import jax
import jax.numpy as jnp
from jax import lax
from jax.experimental import pallas as pl
from jax.experimental.pallas import tpu as pltpu
from jax.experimental.pallas import tpu_sc as plsc

N = 10000
E = 320000
D = 128
DE = 4
F = 128
R = 2 * E

NC, NS, L = 2, 16, 16
NW = NC * NS
BATCH = 128
NCHUNK = 4
EC = E // NCHUNK
RC = 2 * EC
NB_TOT = RC // BATCH
NB_BASE = NB_TOT // NW
NB_EXTRA = NB_TOT - NB_BASE * NW
SC_ITERS = NB_BASE + 1
NSLOT = 4
NSLOT_S = 2
IDX_CAP = SC_ITERS * BATCH
IDX_PAD = (NB_TOT + 1) * BATCH
ROWS_PT = 624
ROWS_TAIL = N - NS * ROWS_PT

_LN_EPS = 1e-6


def _ln(x, g, b):
    mu = jnp.mean(x, axis=-1, keepdims=True)
    xc = x - mu
    var = jnp.mean(xc * xc, axis=-1, keepdims=True)
    inv = lax.rsqrt(var + _LN_EPS)
    return xc * inv * g + b


def _stage1_body(eps_ref, n_ref, sW1_ref, sb1_ref, sg1_ref, sbe1_ref,
                 sW2_ref, sb2_ref, sg2_ref, sbe2_ref, W1a_ref, mb1_ref,
                 h_ref, a_ref):
    n = n_ref[...]
    x = jnp.maximum(jnp.dot(n, sW1_ref[...], preferred_element_type=jnp.float32)
                    + sb1_ref[...], 0.0)
    x = _ln(x, sg1_ref[...], sbe1_ref[...])
    x = jnp.maximum(jnp.dot(x, sW2_ref[...], preferred_element_type=jnp.float32)
                    + sb2_ref[...], 0.0)
    x = _ln(x, sg2_ref[...], sbe2_ref[...])
    h_ref[...] = x * (1.0 + eps_ref[0, 0])
    a_ref[...] = (jnp.dot(n, W1a_ref[...], preferred_element_type=jnp.float32)
                  + mb1_ref[...])


def _stage1(nodes, eps2, sW1, sb1, sg1, sbe1, sW2, sb2, sg2, sbe2, W1a, mb1):
    BN = 2000
    row = pl.BlockSpec((BN, D), lambda i: (i, 0))
    mat = pl.BlockSpec((D, F), lambda i: (0, 0))
    vec = pl.BlockSpec((1, F), lambda i: (0, 0))
    scl = pl.BlockSpec((1, 1), lambda i: (0, 0))
    return pl.pallas_call(
        _stage1_body,
        grid=(N // BN,),
        in_specs=[scl, row, mat, vec, vec, vec, mat, vec, vec, vec, mat, vec],
        out_specs=[row, row],
        out_shape=[jax.ShapeDtypeStruct((N, F), jnp.float32)] * 2,
    )(eps2, nodes, sW1, sb1, sg1, sbe1, sW2, sb2, sg2, sbe2, W1a, mb1)


def _worker_range(wid):
    start = wid * NB_BASE + jnp.minimum(wid, NB_EXTRA)
    nb = jnp.where(wid < NB_EXTRA, NB_BASE + 1, NB_BASE)
    return start, nb


def _sc_gather(table, idx):
    mesh = plsc.VectorSubcoreMesh(core_axis_name="c", subcore_axis_name="s")

    @functools.partial(
        pl.kernel, mesh=mesh,
        out_type=jax.ShapeDtypeStruct((RC, F), jnp.float32),
        scratch_types=[
            pltpu.VMEM((IDX_CAP,), jnp.int32),
            pltpu.VMEM((NSLOT, BATCH, F), jnp.float32),
            pltpu.SemaphoreType.DMA,
            pltpu.SemaphoreType.DMA,
            pltpu.SemaphoreType.DMA,
            pltpu.SemaphoreType.DMA,
            pltpu.SemaphoreType.DMA,
            pltpu.SemaphoreType.DMA,
            pltpu.SemaphoreType.DMA,
            pltpu.SemaphoreType.DMA,
        ],
    )
    def k(table_hbm, idx_hbm, out_hbm, idx_v, rows_v,
          sg0, sg1, sg2, sg3, so0, so1, so2, so3):
        wid = lax.axis_index("s") * NC + lax.axis_index("c")
        start, nb = _worker_range(wid)
        semg = [sg0, sg1, sg2, sg3]
        semo = [so0, so1, so2, so3]
        pltpu.sync_copy(idx_hbm.at[pl.ds(start * BATCH, IDX_CAP)], idx_v)

        def fire(slot, i):
            @pl.when(i < nb)
            def _():
                pltpu.async_copy(
                    table_hbm.at[idx_v.at[pl.ds(i * BATCH, BATCH)]],
                    rows_v.at[slot], semg[slot])

        for j in range(NSLOT):
            fire(j, j)

        def body(g, carry):
            i0 = g * NSLOT
            for j in range(NSLOT):
                i = i0 + j

                @pl.when(i < nb)
                def _(i=i, j=j):
                    pltpu.make_async_copy(
                        table_hbm.at[idx_v.at[pl.ds(i * BATCH, BATCH)]],
                        rows_v.at[j], semg[j]).wait()
                    off = (start + i) * BATCH
                    pltpu.async_copy(rows_v.at[j],
                                     out_hbm.at[pl.ds(off, BATCH)], semo[j])
                    pltpu.make_async_copy(
                        rows_v.at[j], out_hbm.at[pl.ds(off, BATCH)],
                        semo[j]).wait()
                    fire(j, i + NSLOT)

            return carry

        lax.fori_loop(0, -(-SC_ITERS // NSLOT), body, 0)

    return k(table, idx)


def _stage3_body(g_ref, e_ref, w1b_ref, g1_ref, be1_ref,
                 w2_ref, b2_ref, g2_ref, be2_ref, out_ref):
    ep = jnp.dot(e_ref[...], w1b_ref[...], preferred_element_type=jnp.float32)
    x = jnp.maximum(g_ref[...] + ep, 0.0)
    x = _ln(x, g1_ref[...], be1_ref[...])
    x = jnp.maximum(jnp.dot(x, w2_ref[...], preferred_element_type=jnp.float32)
                    + b2_ref[...], 0.0)
    x = _ln(x, g2_ref[...], be2_ref[...])
    out_ref[...] = x


def _stage3(G, edges, W1b, g1, be1, W2, b2, g2, be2, c):
    BE = 5000
    nE = EC // BE
    row = pl.BlockSpec((BE, F), lambda i: (lax.rem(i, 2) * nE + i // 2, 0))
    erow = pl.BlockSpec((BE, DE), lambda i: (c * nE + i // 2, 0))
    mat = pl.BlockSpec((F, F), lambda i: (0, 0))
    w1b = pl.BlockSpec((DE, F), lambda i: (0, 0))
    vec = pl.BlockSpec((1, F), lambda i: (0, 0))
    return pl.pallas_call(
        _stage3_body,
        grid=(RC // BE,),
        in_specs=[row, erow, w1b, vec, vec, mat, vec, vec, vec],
        out_specs=row,
        out_shape=jax.ShapeDtypeStruct((RC, F), jnp.float32),
    )(G, edges, W1b, g1, be1, W2, b2, g2, be2)


def _sc_scatter(msgs, idx, zeros_nf):
    mesh = plsc.VectorSubcoreMesh(core_axis_name="c", subcore_axis_name="s")

    @functools.partial(
        pl.kernel, mesh=mesh,
        out_type=jax.ShapeDtypeStruct((NC * N, F), jnp.float32),
        scratch_types=[
            pltpu.VMEM((NSLOT_S, BATCH), jnp.int32),
            pltpu.VMEM((NSLOT_S, BATCH, F), jnp.float32),
            pltpu.VMEM_SHARED((N, F), jnp.float32),
            pltpu.SemaphoreType.DMA,
            pltpu.SemaphoreType.DMA,
            pltpu.SemaphoreType.DMA,
            pltpu.SemaphoreType.DMA,
        ],
    )
    def k(msgs_hbm, idx_hbm, zeros_hbm, out_hbm, idx_vs, rows_v,
          acc_sh, si0, si1, sm0, sm1):
        cid = lax.axis_index("c")
        sid = lax.axis_index("s")
        wid = sid * NC + cid
        start, nb = _worker_range(wid)
        semi = [si0, si1]
        semm = [sm0, sm1]
        pltpu.sync_copy(zeros_hbm.at[pl.ds(sid * ROWS_PT, ROWS_PT)],
                        acc_sh.at[pl.ds(sid * ROWS_PT, ROWS_PT)])

        @pl.when(sid == NS - 1)
        def _zero_tail():
            pltpu.sync_copy(zeros_hbm.at[pl.ds(NS * ROWS_PT, ROWS_TAIL)],
                            acc_sh.at[pl.ds(NS * ROWS_PT, ROWS_TAIL)])

        plsc.subcore_barrier()

        def fire(slot, i):
            @pl.when(i < nb)
            def _():
                off = (start + i) * BATCH
                pltpu.async_copy(idx_hbm.at[pl.ds(off, BATCH)],
                                 idx_vs.at[slot], semi[slot])
                pltpu.async_copy(msgs_hbm.at[pl.ds(off, BATCH)],
                                 rows_v.at[slot], semm[slot])

        for j in range(NSLOT_S):
            fire(j, j)

        def body(g, carry):
            i0 = g * NSLOT_S
            for j in range(NSLOT_S):
                i = i0 + j

                @pl.when(i < nb)
                def _(i=i, j=j):
                    off = (start + i) * BATCH
                    pltpu.make_async_copy(idx_hbm.at[pl.ds(off, BATCH)],
                                          idx_vs.at[j], semi[j]).wait()
                    pltpu.make_async_copy(msgs_hbm.at[pl.ds(off, BATCH)],
                                          rows_v.at[j], semm[j]).wait()
                    pltpu.sync_copy(rows_v.at[j], acc_sh.at[idx_vs.at[j]],
                                    add=True)
                    fire(j, i + NSLOT_S)

            return carry

        lax.fori_loop(0, -(-SC_ITERS // NSLOT_S), body, 0)
        plsc.subcore_barrier()
        pltpu.sync_copy(acc_sh.at[pl.ds(sid * ROWS_PT, ROWS_PT)],
                        out_hbm.at[pl.ds(cid * N + sid * ROWS_PT, ROWS_PT)])

        @pl.when(sid == NS - 1)
        def _out_tail():
            pltpu.sync_copy(acc_sh.at[pl.ds(NS * ROWS_PT, ROWS_TAIL)],
                            out_hbm.at[pl.ds(cid * N + NS * ROWS_PT, ROWS_TAIL)])

    return k(msgs, idx, zeros_nf)


def _stage5_body(*refs):
    h_ref = refs[0]
    p_refs = refs[1:1 + 2 * NCHUNK]
    (W1_ref, b1_ref, g1_ref, be1_ref, W2_ref, b2_ref, g2_ref, be2_ref,
     out_ref) = refs[1 + 2 * NCHUNK:]
    x = h_ref[...]
    for p in p_refs:
        x = x + p[...]
    x = jnp.maximum(jnp.dot(x, W1_ref[...], preferred_element_type=jnp.float32)
                    + b1_ref[...], 0.0)
    x = _ln(x, g1_ref[...], be1_ref[...])
    x = jnp.maximum(jnp.dot(x, W2_ref[...], preferred_element_type=jnp.float32)
                    + b2_ref[...], 0.0)
    x = _ln(x, g2_ref[...], be2_ref[...])
    out_ref[...] = x


def _stage5(h, parts, W1, b1, g1, be1, W2, b2, g2, be2):
    BN = 2000
    row = pl.BlockSpec((BN, F), lambda i: (i, 0))
    p0 = pl.BlockSpec((BN, F), lambda i: (i, 0))
    p1 = pl.BlockSpec((BN, F), lambda i: (i + N // BN, 0))
    mat = pl.BlockSpec((F, F), lambda i: (0, 0))
    vec = pl.BlockSpec((1, F), lambda i: (0, 0))
    part_args = [p for part in parts for p in (part, part)]
    part_specs = [s for _ in parts for s in (p0, p1)]
    return pl.pallas_call(
        _stage5_body,
        grid=(N // BN,),
        in_specs=[row] + part_specs
                 + [mat, vec, vec, vec, mat, vec, vec, vec],
        out_specs=row,
        out_shape=jax.ShapeDtypeStruct((N, F), jnp.float32),
    )(h, *part_args, W1, b1, g1, be1, W2, b2, g2, be2)


def kernel(nodes, edges, eps_const,
           msg_W1, msg_b1, msg_g1, msg_be1, msg_W2, msg_b2, msg_g2, msg_be2,
           self_W1, self_b1, self_g1, self_be1, self_W2, self_b2, self_g2,
           self_be2, node_W1, node_b1, node_g1, node_be1, node_W2, node_b2,
           node_g2, node_be2, senders, receivers):
    W1a = msg_W1[:D]
    W1b = msg_W1[D:]
    pad = jnp.zeros((IDX_PAD - RC,), jnp.int32)
    eps2 = eps_const.reshape(1, 1)
    zeros_nf = jnp.zeros((N, F), jnp.float32)
    v = lambda a: a.reshape(1, F)

    h_scaled, A = _stage1(nodes, eps2, self_W1, v(self_b1), v(self_g1),
                          v(self_be1), self_W2, v(self_b2), v(self_g2),
                          v(self_be2), W1a, v(msg_b1))
    parts = []
    for c in range(NCHUNK):
        a, b = c * EC, (c + 1) * EC
        s_idx = jnp.concatenate([senders[a:b], receivers[a:b], pad])
        r_idx = jnp.concatenate([receivers[a:b], senders[a:b], pad])
        G = _sc_gather(A, s_idx)
        msgs = _stage3(G, edges, W1b, v(msg_g1), v(msg_be1), msg_W2,
                       v(msg_b2), v(msg_g2), v(msg_be2), c)
        parts.append(_sc_scatter(msgs, r_idx, zeros_nf))
    out = _stage5(h_scaled, parts, node_W1, v(node_b1), v(node_g1),
                  v(node_be1), node_W2, v(node_b2), v(node_g2), v(node_be2))
    return out
